# Initial kernel scaffold; baseline (speedup 1.0000x reference)
import jax
import jax.numpy as jnp
from jax.experimental import pallas as pl


def kernel(feats, labels, bank):
    raise NotImplementedError("write your pallas kernel here")


# speedup vs baseline: 44.5996x; 44.5996x over previous
import jax
import jax.numpy as jnp
from jax import lax
from jax.experimental import pallas as pl
from jax.experimental.pallas import tpu as pltpu
from jax.experimental.pallas import tpu_sc as plsc

NC = 2
NS = 16
NW = NC * NS
L = 16

C = 1000
CP = 1024
Q = 256
D = 128
B = 16384
CHUNK = 128
ROWS_PER_W = B // NW
NGROUP = ROWS_PER_W // CHUNK
SLICE = CP // NS

_MESH = plsc.VectorSubcoreMesh(core_axis_name="c", subcore_axis_name="s",
                               num_cores=NC, num_subcores=NS)


def _norm_tc_body(x_ref, o_ref):
    x = x_ref[...]
    nr = jnp.sqrt(jnp.sum(x * x, axis=1, keepdims=True))
    o_ref[...] = x / jnp.maximum(nr, 1e-12)


def _normalize_tc(feats):
    grid = 8
    rows = B // grid
    return pl.pallas_call(
        _norm_tc_body,
        grid=(grid,),
        in_specs=[pl.BlockSpec((rows, D), lambda i: (i, 0))],
        out_specs=pl.BlockSpec((rows, D), lambda i: (i, 0)),
        out_shape=jax.ShapeDtypeStruct((B, D), jnp.float32),
    )(feats)


def _scatter_sc_body(fnorm_hbm, idx_hbm, zeros_hbm,
                     partial_hbm, ncnt_hbm,
                     fbuf, idxb, onesb, acc, cntacc):
    c = lax.axis_index("c")
    s = lax.axis_index("s")
    wid = c * NS + s

    def fill(i, _):
        def fcol(j, _):
            onesb[i, pl.ds(j * L, L)] = jnp.ones((L,), jnp.float32)
            return 0
        lax.fori_loop(0, D // L, fcol, 0)
        return 0
    lax.fori_loop(0, CHUNK, fill, 0)

    pltpu.sync_copy(zeros_hbm, acc.at[pl.ds(s * SLICE, SLICE)])
    pltpu.sync_copy(zeros_hbm, cntacc.at[pl.ds(s * SLICE, SLICE)])
    plsc.subcore_barrier()

    base = wid * ROWS_PER_W
    for j in range(NGROUP):
        pltpu.sync_copy(idx_hbm.at[pl.ds(base + j * CHUNK, CHUNK)],
                        idxb.at[j])
        pltpu.sync_copy(fnorm_hbm.at[pl.ds(base + j * CHUNK, CHUNK)], fbuf)
        pltpu.sync_copy(fbuf, acc.at[idxb.at[j]], add=True)
        pltpu.sync_copy(onesb, cntacc.at[idxb.at[j]], add=True)

    plsc.subcore_barrier()
    pltpu.sync_copy(acc.at[pl.ds(s * SLICE, SLICE)],
                    partial_hbm.at[c].at[pl.ds(s * SLICE, SLICE)])
    pltpu.sync_copy(cntacc.at[pl.ds(s * SLICE, SLICE)],
                    ncnt_hbm.at[c].at[pl.ds(s * SLICE, SLICE)])


def _scatter_sc(fnorm, idx):
    f = pl.kernel(
        _scatter_sc_body,
        out_type=(
            jax.ShapeDtypeStruct((NC, CP, D), jnp.float32),
            jax.ShapeDtypeStruct((NC, CP, D), jnp.float32),
        ),
        mesh=_MESH,
        scratch_types=[
            pltpu.VMEM((CHUNK, D), jnp.float32),
            pltpu.VMEM((NGROUP, CHUNK), jnp.int32),
            pltpu.VMEM((CHUNK, D), jnp.float32),
            pltpu.VMEM_SHARED((CP, D), jnp.float32),
            pltpu.VMEM_SHARED((CP, D), jnp.float32),
        ],
    )
    zeros = jnp.zeros((SLICE, D), jnp.float32)
    return f(fnorm, idx, zeros)


def _rank_tc_body(lab_ref, n_ref, idx_ref, hist_ref):
    w = pl.program_id(0)

    @pl.when(w == 0)
    def _():
        def zh(i, _):
            hist_ref[i] = 0
            return 0
        lax.fori_loop(0, CP, zh, 0)

    def step(i, _):
        cl = lab_ref[0, 0, i]
        r = hist_ref[cl]
        hist_ref[cl] = r + 1
        keep = r >= n_ref[cl] - Q
        idx_ref[0, 0, i] = jnp.where(keep, cl, C + (cl % 8))
        return 0
    lax.fori_loop(0, ROWS_PER_W, step, 0)


def _rank_tc(labels, n):
    idx = pl.pallas_call(
        _rank_tc_body,
        grid=(NW,),
        in_specs=[
            pl.BlockSpec((1, 1, ROWS_PER_W), lambda w: (w, 0, 0),
                         memory_space=pltpu.SMEM),
            pl.BlockSpec(memory_space=pltpu.SMEM),
        ],
        out_specs=pl.BlockSpec((1, 1, ROWS_PER_W), lambda w: (w, 0, 0),
                               memory_space=pltpu.SMEM),
        out_shape=jax.ShapeDtypeStruct((NW, 1, ROWS_PER_W), jnp.int32),
        scratch_shapes=[pltpu.SMEM((CP,), jnp.int32)],
    )(labels.reshape(NW, 1, ROWS_PER_W), n)
    return idx.reshape(B)


def _final_tc_body(p_ref, n_ref, proto_ref, valid_ref):
    p = p_ref[0] + p_ref[1]
    n = n_ref[0, :, 0] + n_ref[1, :, 0]
    cnt = jnp.minimum(n, float(Q))
    denom = jnp.maximum(cnt, 1.0)
    proto = p / denom[:, None]
    pn = jnp.sqrt(jnp.sum(proto * proto, axis=1, keepdims=True))
    proto_ref[...] = proto / jnp.maximum(pn, 1e-12)
    valid_ref[...] = (n > 0.0).astype(jnp.int32)[:, None]


def _finalize_tc(partial, ncnt):
    return pl.pallas_call(
        _final_tc_body,
        out_shape=(
            jax.ShapeDtypeStruct((CP, D), jnp.float32),
            jax.ShapeDtypeStruct((CP, 1), jnp.int32),
        ),
    )(partial, ncnt)


def kernel(feats, labels, bank):
    del bank
    labels = labels.astype(jnp.int32)
    fnorm = _normalize_tc(feats.astype(jnp.float32))
    partial, ncnt = _scatter_sc(fnorm, labels)
    n = (ncnt[0, :, 0] + ncnt[1, :, 0]).astype(jnp.int32)

    def common(_):
        return partial

    def rare(_):
        idx2 = _rank_tc(labels, n)
        partial2, _ = _scatter_sc(fnorm, idx2)
        return partial2

    partial_f = lax.cond(jnp.any(n > Q), rare, common, operand=None)
    proto, valid = _finalize_tc(partial_f, ncnt)
    return proto[:C], valid[:C, 0].astype(bool)
